# baseline (speedup 1.0000x reference)
import functools

import jax
import jax.numpy as jnp
from jax import lax
from jax.experimental import pallas as pl
from jax.experimental.pallas import tpu as pltpu
from jax.experimental.pallas import tpu_sc as plsc

N_PIDS = 100000
FEAT = 256
BATCH = 4096
NSUB = 16
CHUNK = BATCH // NSUB
HALF = 128
L = 16
NCH = CHUNK // L

MAP_SIZE = N_PIDS + 8
TRASH = BATCH
ACC_ROWS = BATCH + 8
CNT_ROWS = BATCH + 8
LTRASH = CHUNK
BIG = 1 << 20

_mesh = plsc.VectorSubcoreMesh(core_axis_name="c", subcore_axis_name="s")


@functools.partial(
    pl.kernel, out_type=(), mesh=_mesh,
    scratch_types=[
        pltpu.VMEM((HALF,), jnp.int32),
        pltpu.VMEM((HALF,), jnp.int32),
        pltpu.VMEM((HALF,), jnp.int32),
        pltpu.VMEM((HALF,), jnp.int32),
        pltpu.VMEM((HALF,), jnp.int32),
    ],
)
def _k_race(lbl_hbm, lblbuf_ref, map_ref, lbl_a, lbl_b, idx_a, idx_b, val_v):
  c = lax.axis_index("c")
  s = lax.axis_index("s")

  @pl.when(c == 0)
  def _():
    base = s * CHUNK
    lanes = lax.iota(jnp.int32, L)
    for j, lbl_j, idx_j in ((0, lbl_a, idx_a), (1, lbl_b, idx_b)):
      pltpu.sync_copy(lbl_hbm.at[pl.ds(base + j * HALF, HALF)], lbl_j)
      for k in range(HALF // L):
        col = pl.ds(k * L, L)
        idx_j[col] = lanes + (base + j * HALF + k * L)
        val_v[col] = lbl_j[col]
      pltpu.sync_copy(idx_j, map_ref.at[lbl_j])
      pltpu.sync_copy(val_v, lblbuf_ref.at[lbl_j])


@functools.partial(
    pl.kernel, out_type=(), mesh=_mesh,
    scratch_types=[
        pltpu.VMEM((HALF,), jnp.int32),
        pltpu.VMEM((HALF,), jnp.int32),
        pltpu.VMEM((HALF,), jnp.int32),
        pltpu.VMEM((HALF,), jnp.int32),
        pltpu.VMEM((HALF,), jnp.int32),
        pltpu.VMEM((HALF,), jnp.int32),
        pltpu.VMEM((HALF,), jnp.int32),
        pltpu.VMEM((HALF,), jnp.int32),
        pltpu.VMEM((CHUNK,), jnp.int32),
        pltpu.VMEM((CHUNK,), jnp.int32),
        pltpu.VMEM((CHUNK,), jnp.float32),
        pltpu.VMEM((CHUNK + 1, 128), jnp.float32),
        pltpu.VMEM((CHUNK + 1, FEAT), jnp.float32),
    ],
)
def _k_rep(feat_hbm, lbl_hbm, map_ref, acc_ref, cnt_ref,
           rem_ref, mslot_ref, mmult_ref,
           lbl_a, lbl_b, idx_a, idx_b, slot0_a, slot0_b, tgt_a, tgt_b,
           tgtloc_v, mslot_v, mmult_v, multw_v, feat_v):
  c = lax.axis_index("c")
  s = lax.axis_index("s")

  @pl.when(c == 0)
  def _():
    base = s * CHUNK
    lanes = lax.iota(jnp.int32, L)
    halves = ((0, lbl_a, idx_a, slot0_a, tgt_a),
              (1, lbl_b, idx_b, slot0_b, tgt_b))
    for j, lbl_j, idx_j, slot0_j, _ in halves:
      pltpu.sync_copy(lbl_hbm.at[pl.ds(base + j * HALF, HALF)], lbl_j)
      pltpu.sync_copy(feat_hbm.at[pl.ds(base + j * HALF, HALF)],
                      feat_v.at[pl.ds(j * HALF, HALF)])
      for k in range(HALF // L):
        col = pl.ds(k * L, L)
        idx_j[col] = lanes + (base + j * HALF + k * L)
      pltpu.sync_copy(map_ref.at[lbl_j], slot0_j)

    def _dedup_half(h):
      @pl.loop(0, NCH // 2)
      def _chunk(tc):
        roff = h * HALF + tc * L
        sc = slot0_a[pl.ds(tc * L, L)] if h == 0 else slot0_b[pl.ds(tc * L, L)]
        tvec = jnp.zeros((L,), jnp.int32)
        mvec = jnp.zeros((L,), jnp.float32)
        for i in range(L):
          r = roff + i
          y = lax.broadcast(sc[i], (L,))
          rvec = lax.broadcast(r, (L,))
          minv = jnp.full((L,), BIG, jnp.int32)
          matchv = jnp.zeros((L,), jnp.int32)
          for t2 in range(NCH):
            if t2 < NCH // 2:
              oc = slot0_a[pl.ds(t2 * L, L)]
            else:
              oc = slot0_b[pl.ds((t2 - NCH // 2) * L, L)]
            pos = lanes + (t2 * L)
            m = oc == y
            matchv = matchv + jnp.where(m, jnp.int32(1), jnp.int32(0))
            minv = jnp.minimum(
                minv, jnp.where(jnp.logical_and(m, pos < rvec), pos, BIG))
          fmin = minv[0]
          mtot = matchv[0]
          for q in range(1, L):
            fmin = jnp.minimum(fmin, minv[q])
            mtot = mtot + matchv[q]
          first = jnp.minimum(fmin, r)
          mult = mtot.astype(jnp.float32)
          isfirst = first == r
          lanesel = lanes == i
          tvec = jnp.where(
              lanesel,
              lax.broadcast(jnp.where(isfirst, jnp.int32(LTRASH), first), (L,)),
              tvec)
          mvec = jnp.where(lanesel, lax.broadcast(mult, (L,)), mvec)
          for kk in range(128 // L):
            multw_v[r, pl.ds(kk * L, L)] = lax.broadcast(mult, (L,))
        tgtloc_v[pl.ds(roff, L)] = tvec
        mmult_v[pl.ds(roff, L)] = mvec

    _dedup_half(0)
    _dedup_half(1)

    @pl.loop(0, NCH)
    def _combine(tc):
      tv = tgtloc_v[pl.ds(tc * L, L)]
      for i in range(L):
        r = tc * L + i
        tgt = tv[i]
        for k in range(FEAT // L):
          col = pl.ds(k * L, L)
          feat_v[tgt, col] = feat_v[tgt, col] + feat_v[r, col]

    for j, lbl_j, idx_j, slot0_j, tgt_j in halves:
      for k in range(HALF // L):
        col = pl.ds(k * L, L)
        loc = tgtloc_v[pl.ds(j * HALF + k * L, L)]
        isfirst = loc == LTRASH
        inr = jnp.logical_and(slot0_j[col] >= base,
                              slot0_j[col] < base + CHUNK)
        win = jnp.logical_and(inr, isfirst)
        tgt_j[col] = jnp.where(win, slot0_j[col], TRASH)
        mslot_v[pl.ds(j * HALF + k * L, L)] = jnp.where(
            isfirst, jnp.where(inr, BIG, slot0_j[col]), BIG)
      pltpu.sync_copy(feat_v.at[pl.ds(j * HALF, HALF)], acc_ref.at[tgt_j])
      pltpu.sync_copy(multw_v.at[pl.ds(j * HALF, HALF)], cnt_ref.at[tgt_j])
    pltpu.sync_copy(feat_v.at[pl.ds(0, CHUNK)],
                    rem_ref.at[pl.ds(base, CHUNK)])
    pltpu.sync_copy(mslot_v, mslot_ref.at[pl.ds(base, CHUNK)])
    pltpu.sync_copy(mmult_v, mmult_ref.at[pl.ds(base, CHUNK)])


@functools.partial(
    pl.kernel, out_type=(), mesh=_mesh,
    scratch_types=[
        pltpu.VMEM((CHUNK,), jnp.int32),
        pltpu.VMEM((CHUNK,), jnp.float32),
        pltpu.VMEM((FEAT,), jnp.float32),
        pltpu.VMEM((CHUNK, 128), jnp.float32),
        pltpu.VMEM((CHUNK, FEAT), jnp.float32),
    ],
)
def _k_merge(acc_ref, cnt_ref, rem_ref, mslot_ref, mmult_ref,
             mslot_v, mmult_v, row_v, cntb_v, accb_v):
  c = lax.axis_index("c")
  s = lax.axis_index("s")

  @pl.when(c == 0)
  def _():
    lo = s * CHUNK
    pltpu.sync_copy(acc_ref.at[pl.ds(lo, CHUNK)], accb_v)
    pltpu.sync_copy(cnt_ref.at[pl.ds(lo, CHUNK)], cntb_v)

    @pl.loop(0, NSUB)
    def _region(reg):
      pltpu.sync_copy(mslot_ref.at[pl.ds(reg * CHUNK, CHUNK)], mslot_v)
      pltpu.sync_copy(mmult_ref.at[pl.ds(reg * CHUNK, CHUNK)], mmult_v)

      @pl.loop(0, NCH)
      def _chunk(tc):
        sl = mslot_v[pl.ds(tc * L, L)]
        mu = mmult_v[pl.ds(tc * L, L)]
        for i in range(L):
          slot = sl[i]
          mine = jnp.logical_and(slot >= lo, slot < lo + CHUNK)

          @pl.when(mine)
          def _pull():
            p = reg * CHUNK + tc * L + i
            pltpu.sync_copy(rem_ref.at[p], row_v)
            lrow = slot - lo
            for k in range(FEAT // L):
              col = pl.ds(k * L, L)
              accb_v[lrow, col] = accb_v[lrow, col] + row_v[col]
            cntb_v[lrow, pl.ds(0, L)] = (cntb_v[lrow, pl.ds(0, L)]
                                         + lax.broadcast(mu[i], (L,)))

    pltpu.sync_copy(accb_v, acc_ref.at[pl.ds(lo, CHUNK)])
    pltpu.sync_copy(cntb_v, cnt_ref.at[pl.ds(lo, CHUNK)])


@functools.partial(
    pl.kernel, out_type=(), mesh=_mesh,
    scratch_types=[
        pltpu.VMEM((HALF,), jnp.int32),
        pltpu.VMEM((HALF,), jnp.int32),
        pltpu.VMEM((HALF,), jnp.int32),
        pltpu.VMEM((HALF,), jnp.int32),
        pltpu.VMEM((HALF, 128), jnp.float32),
        pltpu.VMEM((HALF, FEAT), jnp.float32),
    ],
)
def _k_final(lbl_hbm, map_ref, acc_ref, cnt_ref, proto_ref,
             lbl_a, lbl_b, slot0_a, slot0_b, cntv_v, rows_v):
  c = lax.axis_index("c")
  s = lax.axis_index("s")

  @pl.when(c == 0)
  def _():
    base = s * CHUNK
    for j, lbl_j, slot0_j in ((0, lbl_a, slot0_a), (1, lbl_b, slot0_b)):
      pltpu.sync_copy(lbl_hbm.at[pl.ds(base + j * HALF, HALF)], lbl_j)
      pltpu.sync_copy(map_ref.at[lbl_j], slot0_j)
      pltpu.sync_copy(acc_ref.at[slot0_j], rows_v)
      pltpu.sync_copy(cnt_ref.at[slot0_j], cntv_v)

      @pl.loop(0, HALF)
      def _scale(r):
        rec = jnp.full((L,), 1.0, jnp.float32) / cntv_v[r, pl.ds(0, L)]
        for k in range(FEAT // L):
          col = pl.ds(k * L, L)
          rows_v[r, col] = rows_v[r, col] * rec

      pltpu.sync_copy(rows_v, proto_ref.at[lbl_j])


def kernel(features, labels, prototype_feature, label_buf):
  lbl1d = labels.astype(jnp.int32).reshape(BATCH)
  proto_ref = jax.new_ref(prototype_feature)
  lblbuf_ref = jax.new_ref(label_buf.astype(jnp.int32))
  map_ref = jax.new_ref(jnp.zeros((MAP_SIZE,), jnp.int32))
  acc_ref = jax.new_ref(jnp.zeros((ACC_ROWS, FEAT), jnp.float32))
  cnt_ref = jax.new_ref(jnp.zeros((CNT_ROWS, 128), jnp.float32))
  rem_ref = jax.new_ref(jnp.zeros((BATCH, FEAT), jnp.float32))
  mslot_ref = jax.new_ref(jnp.zeros((BATCH,), jnp.int32))
  mmult_ref = jax.new_ref(jnp.zeros((BATCH,), jnp.float32))
  _k_race(lbl1d, lblbuf_ref, map_ref)
  _k_rep(features, lbl1d, map_ref, acc_ref, cnt_ref,
         rem_ref, mslot_ref, mmult_ref)
  _k_merge(acc_ref, cnt_ref, rem_ref, mslot_ref, mmult_ref)
  _k_final(lbl1d, map_ref, acc_ref, cnt_ref, proto_ref)
  return jax.freeze(proto_ref), jax.freeze(lblbuf_ref).astype(label_buf.dtype)
